# Initial kernel scaffold; baseline (speedup 1.0000x reference)
import functools

import jax
import jax.numpy as jnp
from jax import lax
from jax.experimental import pallas as pl
from jax.experimental.pallas import tpu as pltpu
from jax.experimental.pallas import tpu_sc as plsc

N = 10000
H = 128
E = 160000
CW = 16

NCORES = 2
NTILES = 16
C = 80
EPT = E // NTILES
NCHUNK = EPT // C
RPT = N // NTILES


def _sc_agg_body(table, src4, dst4, zrows, zcnt, ones,
                 sums_out, cnt_out,
                 srcidx_v, dstidx_v, rows_v, ones_v, ssum, scnt, sem):
    cid = lax.axis_index("c")
    sid = lax.axis_index("s")
    r0 = sid * RPT
    pltpu.sync_copy(zrows.at[pl.ds(r0, RPT)], ssum.at[pl.ds(r0, RPT)])
    pltpu.sync_copy(zcnt.at[pl.ds(r0, RPT)], scnt.at[pl.ds(r0, RPT)])
    pltpu.sync_copy(ones, ones_v)
    pltpu.sync_copy(src4.at[cid, sid], srcidx_v)
    pltpu.sync_copy(dst4.at[cid, sid], dstidx_v)
    plsc.subcore_barrier()

    def body(j, carry):
        pltpu.async_copy(table.at[srcidx_v.at[j]], rows_v, sem).wait()
        pltpu.sync_copy(rows_v, ssum.at[dstidx_v.at[j]], add=True)
        pltpu.sync_copy(ones_v, scnt.at[dstidx_v.at[j]], add=True)
        return carry

    lax.fori_loop(0, NCHUNK, body, 0)
    plsc.subcore_barrier()
    pltpu.sync_copy(ssum.at[pl.ds(r0, RPT)], sums_out.at[cid, pl.ds(r0, RPT)])
    pltpu.sync_copy(scnt.at[pl.ds(r0, RPT)], cnt_out.at[cid, pl.ds(r0, RPT)])


_sc_agg = functools.partial(
    pl.kernel,
    out_type=[
        jax.ShapeDtypeStruct((NCORES, N, H), jnp.float32),
        jax.ShapeDtypeStruct((NCORES, N, CW), jnp.float32),
    ],
    mesh=plsc.VectorSubcoreMesh(core_axis_name="c", subcore_axis_name="s"),
    scratch_types=[
        pltpu.VMEM((NCHUNK, C), jnp.int32),
        pltpu.VMEM((NCHUNK, C), jnp.int32),
        pltpu.VMEM((C, H), jnp.float32),
        pltpu.VMEM((C, CW), jnp.float32),
        pltpu.VMEM_SHARED((N, H), jnp.float32),
        pltpu.VMEM_SHARED((N, CW), jnp.float32),
        pltpu.SemaphoreType.DMA,
    ],
)(_sc_agg_body)


def _dense_body(sums_ref, cnt_ref, Wr_ref, Wp_ref, br_ref, bp_ref,
                W1_ref, b1_ref, g_ref, beta_ref, W2_ref, b2_ref, out_ref):
    cnt_r = cnt_ref[0, :, 0:1]
    cnt_p = cnt_ref[1, :, 0:1]
    mean_r = sums_ref[0] / jnp.maximum(cnt_r, 1.0)
    mean_p = sums_ref[1] / jnp.maximum(cnt_p, 1.0)
    has_r = (cnt_r > 0.0).astype(jnp.float32)
    has_p = (cnt_p > 0.0).astype(jnp.float32)
    dot = functools.partial(jnp.dot, precision=jax.lax.Precision.HIGHEST,
                            preferred_element_type=jnp.float32)
    agg = (dot(mean_r, Wr_ref[...]) + dot(mean_p, Wp_ref[...])
           + br_ref[...] * has_r + bp_ref[...] * has_p)
    h = jnp.maximum(dot(agg, W1_ref[...]) + b1_ref[...], 0.0)
    m = jnp.mean(h, axis=0, keepdims=True)
    v = jnp.mean((h - m) * (h - m), axis=0, keepdims=True)
    h = (h - m) / jnp.sqrt(v + 1e-5) * g_ref[...] + beta_ref[...]
    out_ref[...] = dot(h, W2_ref[...]) + b2_ref[...]


_dense = pl.pallas_call(
    _dense_body,
    out_shape=jax.ShapeDtypeStruct((N, H), jnp.float32),
)


def _stage(sums, cnt, Wr, Wp, br, bp, W1, b1, g, beta, W2, b2):
    return _dense(sums, cnt, Wr, Wp, br.reshape(1, H), bp.reshape(1, H),
                  W1, b1.reshape(1, H), g.reshape(1, H), beta.reshape(1, H),
                  W2, b2.reshape(1, H))


def kernel(nodes_input, W_r2e, b_r2e, W_p2e, b_p2e, ra_W1, ra_b1, ra_g,
           ra_beta, ra_W2, ra_b2, W_r2r, b_r2r, W_r2p, b_r2p, ma_W1, ma_b1,
           ma_g, ma_beta, ma_W2, ma_b2, edges):
    nodes = nodes_input.reshape(N, H)
    src1 = jnp.stack([edges[0, 0], edges[1, 0]]).reshape(NCORES, NTILES, NCHUNK, C)
    dst1 = jnp.stack([edges[0, 1], edges[1, 1]]).reshape(NCORES, NTILES, NCHUNK, C)
    src2 = jnp.stack([edges[2, 0], edges[3, 0]]).reshape(NCORES, NTILES, NCHUNK, C)
    dst2 = jnp.stack([edges[2, 1], edges[3, 1]]).reshape(NCORES, NTILES, NCHUNK, C)
    zrows = jnp.zeros((N, H), jnp.float32)
    zcnt = jnp.zeros((N, CW), jnp.float32)
    ones = jnp.ones((C, CW), jnp.float32)

    sums1, cnt1 = _sc_agg(nodes, src1, dst1, zrows, zcnt, ones)
    rh = _stage(sums1, cnt1, W_r2e, W_p2e, b_r2e, b_p2e,
                ra_W1, ra_b1, ra_g, ra_beta, ra_W2, ra_b2)
    sums2, cnt2 = _sc_agg(rh, src2, dst2, zrows, zcnt, ones)
    out = _stage(sums2, cnt2, W_r2r, W_r2p, b_r2r, b_r2p,
                 ma_W1, ma_b1, ma_g, ma_beta, ma_W2, ma_b2)
    return out[None]


# speedup vs baseline: 3.1718x; 3.1718x over previous
import functools

import jax
import jax.numpy as jnp
from jax import lax
from jax.experimental import pallas as pl
from jax.experimental.pallas import tpu as pltpu
from jax.experimental.pallas import tpu_sc as plsc

N = 10000
H = 128
E = 160000
CW = 16

NCORES = 2
NTILES = 16
C = 80
EPT = E // NTILES
NCHUNK = EPT // C
NP = 10112
RPT = NP // NTILES
EPT2 = 2 * E // NTILES
NCHUNK2 = EPT2 // C
CRPT = 2 * NP // NTILES


def _sc_agg_body(table, src4, dst4, zrows, sums_out,
                 srcidx_v, dstidx_v, rows_v, ssum, sem):
    cid = lax.axis_index("c")
    sid = lax.axis_index("s")
    r0 = sid * RPT
    pltpu.sync_copy(zrows.at[pl.ds(r0, RPT)], ssum.at[pl.ds(r0, RPT)])
    plsc.subcore_barrier()

    def body(j, carry):
        pltpu.sync_copy(src4.at[cid, sid, j], srcidx_v)
        pltpu.sync_copy(dst4.at[cid, sid, j], dstidx_v)
        pltpu.async_copy(table.at[srcidx_v.at[0]], rows_v, sem).wait()
        pltpu.sync_copy(rows_v, ssum.at[dstidx_v.at[0]], add=True)
        return carry

    lax.fori_loop(0, NCHUNK, body, 0)
    plsc.subcore_barrier()
    pltpu.sync_copy(ssum.at[pl.ds(r0, RPT)], sums_out.at[cid, pl.ds(r0, RPT)])


_sc_agg = functools.partial(
    pl.kernel,
    out_type=[jax.ShapeDtypeStruct((NCORES, NP, H), jnp.float32)],
    mesh=plsc.VectorSubcoreMesh(core_axis_name="c", subcore_axis_name="s",
                                num_cores=NCORES, num_subcores=NTILES),
    scratch_types=[
        pltpu.VMEM((1, C), jnp.int32),
        pltpu.VMEM((1, C), jnp.int32),
        pltpu.VMEM((C, H), jnp.float32),
        pltpu.VMEM_SHARED((NP, H), jnp.float32),
        pltpu.SemaphoreType.DMA,
    ],
)(_sc_agg_body)


def _sc_cnt_body(dst4, zrows, ones, cnt_out, dstidx_v, ones_v, scnt, sem):
    cid = lax.axis_index("c")
    sid = lax.axis_index("s")
    r0 = sid * RPT
    pltpu.sync_copy(zrows.at[pl.ds(r0, RPT)], scnt.at[pl.ds(r0, RPT)])
    pltpu.sync_copy(ones, ones_v)
    plsc.subcore_barrier()

    def body(j, carry):
        pltpu.sync_copy(dst4.at[cid, sid, j], dstidx_v)
        pltpu.sync_copy(ones_v, scnt.at[dstidx_v.at[0]], add=True)
        return carry

    lax.fori_loop(0, NCHUNK, body, 0)
    plsc.subcore_barrier()
    pltpu.sync_copy(scnt.at[pl.ds(r0, RPT)], cnt_out.at[cid, pl.ds(r0, RPT)])


_sc_cnt = functools.partial(
    pl.kernel,
    out_type=[jax.ShapeDtypeStruct((NCORES, NP, H), jnp.float32)],
    mesh=plsc.VectorSubcoreMesh(core_axis_name="c", subcore_axis_name="s",
                                num_cores=NCORES, num_subcores=NTILES),
    scratch_types=[
        pltpu.VMEM((1, C), jnp.int32),
        pltpu.VMEM((C, H), jnp.float32),
        pltpu.VMEM_SHARED((NP, H), jnp.float32),
        pltpu.SemaphoreType.DMA,
    ],
)(_sc_cnt_body)


def _dense_body(sums_ref, cntr_ref, cntp_ref, Wr_ref, Wp_ref, br_ref, bp_ref,
                W1_ref, b1_ref, g_ref, beta_ref, W2_ref, b2_ref, out_ref):
    cnt_r = cntr_ref[:N, 0:1]
    cnt_p = cntp_ref[:N, 0:1]
    mean_r = sums_ref[0, :N, :] / jnp.maximum(cnt_r, 1.0)
    mean_p = sums_ref[1, :N, :] / jnp.maximum(cnt_p, 1.0)
    has_r = (cnt_r > 0.0).astype(jnp.float32)
    has_p = (cnt_p > 0.0).astype(jnp.float32)
    dot = functools.partial(jnp.dot, precision=jax.lax.Precision.HIGHEST,
                            preferred_element_type=jnp.float32)
    agg = (dot(mean_r, Wr_ref[...]) + dot(mean_p, Wp_ref[...])
           + br_ref[...] * has_r + bp_ref[...] * has_p)
    h = jnp.maximum(dot(agg, W1_ref[...]) + b1_ref[...], 0.0)
    m = jnp.mean(h, axis=0, keepdims=True)
    v = jnp.mean((h - m) * (h - m), axis=0, keepdims=True)
    h = (h - m) / jnp.sqrt(v + 1e-5) * g_ref[...] + beta_ref[...]
    out_ref[...] = dot(h, W2_ref[...]) + b2_ref[...]


_dense = pl.pallas_call(
    _dense_body,
    out_shape=jax.ShapeDtypeStruct((N, H), jnp.float32),
)


def _stage(sums, cnt_r, cnt_p, Wr, Wp, br, bp, W1, b1, g, beta, W2, b2):
    return _dense(sums, cnt_r, cnt_p, Wr, Wp, br.reshape(1, H),
                  bp.reshape(1, H), W1, b1.reshape(1, H), g.reshape(1, H),
                  beta.reshape(1, H), W2, b2.reshape(1, H))


def kernel(nodes_input, W_r2e, b_r2e, W_p2e, b_p2e, ra_W1, ra_b1, ra_g,
           ra_beta, ra_W2, ra_b2, W_r2r, b_r2r, W_r2p, b_r2p, ma_W1, ma_b1,
           ma_g, ma_beta, ma_W2, ma_b2, edges):
    nodes = nodes_input.reshape(N, H)
    src1 = jnp.stack([edges[0, 0], edges[1, 0]]).reshape(NCORES, NTILES, NCHUNK, 1, C)
    dst1 = jnp.stack([edges[0, 1], edges[1, 1]]).reshape(NCORES, NTILES, NCHUNK, 1, C)
    src2 = jnp.stack([edges[2, 0], edges[3, 0]]).reshape(NCORES, NTILES, NCHUNK, 1, C)
    dst2 = jnp.stack([edges[2, 1], edges[3, 1]]).reshape(NCORES, NTILES, NCHUNK, 1, C)
    zrows = jnp.zeros((NP, H), jnp.float32)
    ones = jnp.ones((C, H), jnp.float32)

    cnt1 = _sc_cnt(dst1, zrows, ones)[0]
    cnt2 = _sc_cnt(dst2, zrows, ones)[0]
    sums1 = _sc_agg(nodes, src1, dst1, zrows)[0]
    rh = _stage(sums1, cnt1[0, :, 0:1], cnt1[1, :, 0:1],
                W_r2e, W_p2e, b_r2e, b_p2e,
                ra_W1, ra_b1, ra_g, ra_beta, ra_W2, ra_b2)
    sums2 = _sc_agg(rh, src2, dst2, zrows)[0]
    out = _stage(sums2, cnt2[0, :, 0:1], cnt2[1, :, 0:1],
                 W_r2r, W_r2p, b_r2r, b_r2p,
                 ma_W1, ma_b1, ma_g, ma_beta, ma_W2, ma_b2)
    return out[None]
